# baseline (speedup 1.0000x reference)
import functools

import jax
import jax.numpy as jnp
from jax import lax
from jax.experimental import pallas as pl
from jax.experimental.pallas import tpu as pltpu
from jax.experimental.pallas import tpu_sc as plsc


def _vq_idx_body(x_ref, w_ref, idx_ref, data_ref, *, imgs_per_step):
    w = w_ref[...]
    w2 = jnp.sum(w * w, axis=1, keepdims=True)
    P = x_ref.shape[2]
    for j in range(imgs_per_step):
        x_t = x_ref[j]
        scores = jax.lax.dot_general(
            w, x_t, (((1,), (0,)), ((), ())),
            preferred_element_type=jnp.float32,
            precision=jax.lax.Precision.DEFAULT,
        )
        d2 = jnp.sum(x_t * x_t, axis=0, keepdims=True)
        dist = d2 - 2.0 * scores + w2
        m = jnp.min(dist, axis=0, keepdims=True)
        iota = jax.lax.broadcasted_iota(jnp.int32, dist.shape, 0)
        idx_ref[pl.ds(j * P, P)] = jnp.min(
            jnp.where(dist <= m, iota, dist.shape[0]), axis=0)
        data_ref[j] = x_t.T


def _make_sc_gather(embed_dim, rows):
    info = plsc.get_sparse_core_info()
    nw = info.num_cores * info.num_subcores
    b_per_w = rows // nw
    mesh = plsc.VectorSubcoreMesh(core_axis_name="c", subcore_axis_name="s")

    @functools.partial(
        pl.kernel, mesh=mesh,
        compiler_params=pltpu.CompilerParams(use_tc_tiling_on_sc=False),
        out_type=jax.ShapeDtypeStruct((rows, embed_dim), jnp.float32),
        scratch_types=[
            pltpu.VMEM((b_per_w,), jnp.int32),
            pltpu.VMEM((b_per_w, embed_dim), jnp.float32),
            pltpu.SemaphoreType.DMA,
        ],
    )
    def gather_k(table_hbm, idx_hbm, out_hbm, idx_v, rows_v, sem):
        wid = lax.axis_index("s") * info.num_cores + lax.axis_index("c")
        base = wid * b_per_w
        pltpu.sync_copy(idx_hbm.at[pl.ds(base, b_per_w)], idx_v)
        pltpu.async_copy(table_hbm.at[idx_v], rows_v, sem).wait()
        pltpu.sync_copy(rows_v, out_hbm.at[pl.ds(base, b_per_w)])

    return gather_k


def kernel(input_data, embed_weights):
    N, C, H, W = input_data.shape
    P = H * W
    rows = N * P
    num_embed = embed_weights.shape[0]
    x = input_data.reshape(N, C, P)
    ips = 2

    def tc_call(x_chunk):
        n = x_chunk.shape[0]
        return pl.pallas_call(
            functools.partial(_vq_idx_body, imgs_per_step=ips),
            grid=(n // ips,),
            in_specs=[
                pl.BlockSpec((ips, C, P), lambda i: (i, 0, 0)),
                pl.BlockSpec((num_embed, C), lambda i: (0, 0)),
            ],
            out_specs=[
                pl.BlockSpec((ips * P,), lambda i: (i,)),
                pl.BlockSpec((ips, P, C), lambda i: (i, 0, 0)),
            ],
            out_shape=[
                jax.ShapeDtypeStruct((n * P,), jnp.int32),
                jax.ShapeDtypeStruct((n, P, C), jnp.float32),
            ],
        )(x_chunk, embed_weights)

    half = N // 2
    gather = _make_sc_gather(C, half * P)
    idx0, data0 = tc_call(x[:half])
    q0 = gather(embed_weights, idx0)
    idx1, data1 = tc_call(x[half:])
    q1 = gather(embed_weights, idx1)
    quantize = jnp.concatenate([q0, q1], axis=0)
    data = jnp.concatenate([data0, data1], axis=0).reshape(rows, C)
    return quantize, quantize, data


# speedup vs baseline: 1.1920x; 1.1920x over previous
import functools

import jax
import jax.numpy as jnp
from jax import lax
from jax.experimental import pallas as pl
from jax.experimental.pallas import tpu as pltpu
from jax.experimental.pallas import tpu_sc as plsc


def _vq_idx_body(x_ref, w_ref, idx_ref, data_ref, *, imgs_per_step):
    w = w_ref[...]
    w2 = jnp.sum(w * w, axis=1, keepdims=True)
    P = x_ref.shape[2]
    for j in range(imgs_per_step):
        x_t = x_ref[j]
        scores = jax.lax.dot_general(
            w, x_t, (((1,), (0,)), ((), ())),
            preferred_element_type=jnp.float32,
            precision=jax.lax.Precision.DEFAULT,
        )
        d2 = jnp.sum(x_t * x_t, axis=0, keepdims=True)
        dist = d2 - 2.0 * scores + w2
        m = jnp.min(dist, axis=0, keepdims=True)
        iota = jax.lax.broadcasted_iota(jnp.int32, dist.shape, 0)
        idx_ref[pl.ds(j * P, P)] = jnp.min(
            jnp.where(dist <= m, iota, dist.shape[0]), axis=0)
        data_ref[j] = x_t.T


_NCH = 4


def _make_sc_gather(embed_dim, rows):
    info = plsc.get_sparse_core_info()
    nw = info.num_cores * info.num_subcores
    b_per_w = rows // nw
    mesh = plsc.VectorSubcoreMesh(core_axis_name="c", subcore_axis_name="s")

    @functools.partial(
        pl.kernel, mesh=mesh,
        compiler_params=pltpu.CompilerParams(use_tc_tiling_on_sc=False),
        out_type=jax.ShapeDtypeStruct((rows, embed_dim), jnp.float32),
        scratch_types=[
            pltpu.VMEM((b_per_w,), jnp.int32),
            pltpu.VMEM((b_per_w, embed_dim), jnp.float32),
        ] + [pltpu.SemaphoreType.DMA] * (2 * _NCH),
    )
    def gather_k(table_hbm, idx_hbm, out_hbm, idx_v, rows_v, *sems):
        gsems, ssems = sems[:_NCH], sems[_NCH:]
        wid = lax.axis_index("s") * info.num_cores + lax.axis_index("c")
        base = wid * b_per_w
        ch = b_per_w // _NCH
        pltpu.sync_copy(idx_hbm.at[pl.ds(base, b_per_w)], idx_v)
        gcops = [
            pltpu.async_copy(
                table_hbm.at[idx_v.at[pl.ds(c * ch, ch)]],
                rows_v.at[pl.ds(c * ch, ch)], gsems[c])
            for c in range(_NCH)
        ]
        scops = []
        for c in range(_NCH):
            gcops[c].wait()
            scops.append(pltpu.async_copy(
                rows_v.at[pl.ds(c * ch, ch)],
                out_hbm.at[pl.ds(base + c * ch, ch)], ssems[c]))
        for cp in scops:
            cp.wait()

    return gather_k


def kernel(input_data, embed_weights):
    N, C, H, W = input_data.shape
    P = H * W
    rows = N * P
    num_embed = embed_weights.shape[0]
    x = input_data.reshape(N, C, P)
    ips = 2

    idx, data = pl.pallas_call(
        functools.partial(_vq_idx_body, imgs_per_step=ips),
        grid=(N // ips,),
        in_specs=[
            pl.BlockSpec((ips, C, P), lambda i: (i, 0, 0)),
            pl.BlockSpec((num_embed, C), lambda i: (0, 0)),
        ],
        out_specs=[
            pl.BlockSpec((ips * P,), lambda i: (i,)),
            pl.BlockSpec((ips, P, C), lambda i: (i, 0, 0)),
        ],
        out_shape=[
            jax.ShapeDtypeStruct((rows,), jnp.int32),
            jax.ShapeDtypeStruct((N, P, C), jnp.float32),
        ],
    )(x, embed_weights)
    data = data.reshape(rows, C)
    quantize = _make_sc_gather(C, rows)(embed_weights, idx)
    return quantize, quantize, data


# speedup vs baseline: 1.2105x; 1.0156x over previous
import functools

import jax
import jax.numpy as jnp
from jax import lax
from jax.experimental import pallas as pl
from jax.experimental.pallas import tpu as pltpu
from jax.experimental.pallas import tpu_sc as plsc


def _vq_idx_body(x_ref, w_ref, idx_ref, data_ref, *, imgs_per_step):
    w = w_ref[...]
    w2 = jnp.sum(w * w, axis=1, keepdims=True)
    P = x_ref.shape[2]
    for j in range(imgs_per_step):
        x_t = x_ref[j]
        scores = jax.lax.dot_general(
            w, x_t, (((1,), (0,)), ((), ())),
            preferred_element_type=jnp.float32,
            precision=jax.lax.Precision.DEFAULT,
        )
        d2 = jnp.sum(x_t * x_t, axis=0, keepdims=True)
        dist = d2 - 2.0 * scores + w2
        m = jnp.min(dist, axis=0, keepdims=True)
        iota = jax.lax.broadcasted_iota(jnp.int32, dist.shape, 0)
        idx_ref[pl.ds(j * P, P)] = jnp.min(
            jnp.where(dist <= m, iota, dist.shape[0]), axis=0)
        data_ref[j] = x_t.T


def _make_sc_gather(embed_dim, rows):
    info = plsc.get_sparse_core_info()
    nw = info.num_cores * info.num_subcores
    b_per_w = rows // nw
    mesh = plsc.VectorSubcoreMesh(core_axis_name="c", subcore_axis_name="s")

    @functools.partial(
        pl.kernel, mesh=mesh,
        compiler_params=pltpu.CompilerParams(use_tc_tiling_on_sc=False),
        out_type=jax.ShapeDtypeStruct((rows, embed_dim), jnp.float32),
        scratch_types=[
            pltpu.VMEM((b_per_w,), jnp.int32),
            pltpu.VMEM((b_per_w, embed_dim), jnp.float32),
        ] + [pltpu.SemaphoreType.DMA],
    )
    def gather_k(table_hbm, idx_hbm, out_hbm, idx_v, rows_v, sem):
        wid = lax.axis_index("s") * info.num_cores + lax.axis_index("c")
        base = wid * b_per_w
        pltpu.sync_copy(idx_hbm.at[pl.ds(base, b_per_w)], idx_v)
        pltpu.async_copy(table_hbm.at[idx_v], rows_v, sem).wait()
        pltpu.sync_copy(rows_v, out_hbm.at[pl.ds(base, b_per_w)])

    return gather_k


def kernel(input_data, embed_weights):
    N, C, H, W = input_data.shape
    P = H * W
    rows = N * P
    num_embed = embed_weights.shape[0]
    x = input_data.reshape(N, C, P)
    ips = 2

    idx, data = pl.pallas_call(
        functools.partial(_vq_idx_body, imgs_per_step=ips),
        grid=(N // ips,),
        in_specs=[
            pl.BlockSpec((ips, C, P), lambda i: (i, 0, 0)),
            pl.BlockSpec((num_embed, C), lambda i: (0, 0)),
        ],
        out_specs=[
            pl.BlockSpec((ips * P,), lambda i: (i,)),
            pl.BlockSpec((ips, P, C), lambda i: (i, 0, 0)),
        ],
        out_shape=[
            jax.ShapeDtypeStruct((rows,), jnp.int32),
            jax.ShapeDtypeStruct((N, P, C), jnp.float32),
        ],
    )(x, embed_weights)
    data = data.reshape(rows, C)
    quantize = _make_sc_gather(C, rows)(embed_weights, idx)
    return quantize, quantize, data


# speedup vs baseline: 1.2860x; 1.0624x over previous
import functools

import jax
import jax.numpy as jnp
from jax import lax
from jax.experimental import pallas as pl
from jax.experimental.pallas import tpu as pltpu
from jax.experimental.pallas import tpu_sc as plsc


def _vq_idx_body(x_ref, w_ref, idx_ref, data_ref, *, imgs_per_step):
    w = w_ref[...]
    w2 = jnp.sum(w * w, axis=1, keepdims=True)
    P = x_ref.shape[2]
    for j in range(imgs_per_step):
        x_t = x_ref[j]
        scores = jax.lax.dot_general(
            w, x_t, (((1,), (0,)), ((), ())),
            preferred_element_type=jnp.float32,
            precision=jax.lax.Precision.DEFAULT,
        )
        d2 = jnp.sum(x_t * x_t, axis=0, keepdims=True)
        dist = d2 - 2.0 * scores + w2
        idx_ref[pl.ds(j * P, P)] = jnp.argmin(dist, axis=0).astype(jnp.int32)
        data_ref[j] = x_t.T


def _make_sc_gather(embed_dim, rows):
    info = plsc.get_sparse_core_info()
    nw = info.num_cores * info.num_subcores
    b_per_w = rows // nw
    mesh = plsc.VectorSubcoreMesh(core_axis_name="c", subcore_axis_name="s")

    @functools.partial(
        pl.kernel, mesh=mesh,
        compiler_params=pltpu.CompilerParams(use_tc_tiling_on_sc=False),
        out_type=jax.ShapeDtypeStruct((rows, embed_dim), jnp.float32),
        scratch_types=[
            pltpu.VMEM((b_per_w,), jnp.int32),
            pltpu.VMEM((b_per_w, embed_dim), jnp.float32),
        ] + [pltpu.SemaphoreType.DMA],
    )
    def gather_k(table_hbm, idx_hbm, out_hbm, idx_v, rows_v, sem):
        wid = lax.axis_index("s") * info.num_cores + lax.axis_index("c")
        base = wid * b_per_w
        pltpu.sync_copy(idx_hbm.at[pl.ds(base, b_per_w)], idx_v)
        pltpu.async_copy(table_hbm.at[idx_v], rows_v, sem).wait()
        pltpu.sync_copy(rows_v, out_hbm.at[pl.ds(base, b_per_w)])

    return gather_k


def kernel(input_data, embed_weights):
    N, C, H, W = input_data.shape
    P = H * W
    rows = N * P
    num_embed = embed_weights.shape[0]
    x = input_data.reshape(N, C, P)
    ips = 2

    idx, data = pl.pallas_call(
        functools.partial(_vq_idx_body, imgs_per_step=ips),
        grid=(N // ips,),
        in_specs=[
            pl.BlockSpec((ips, C, P), lambda i: (i, 0, 0)),
            pl.BlockSpec((num_embed, C), lambda i: (0, 0)),
        ],
        out_specs=[
            pl.BlockSpec((ips * P,), lambda i: (i,)),
            pl.BlockSpec((ips, P, C), lambda i: (i, 0, 0)),
        ],
        out_shape=[
            jax.ShapeDtypeStruct((rows,), jnp.int32),
            jax.ShapeDtypeStruct((N, P, C), jnp.float32),
        ],
    )(x, embed_weights)
    data = data.reshape(rows, C)
    quantize = _make_sc_gather(C, rows)(embed_weights, idx)
    return quantize, quantize, data


# speedup vs baseline: 1.2878x; 1.0014x over previous
import functools

import jax
import jax.numpy as jnp
from jax import lax
from jax.experimental import pallas as pl
from jax.experimental.pallas import tpu as pltpu
from jax.experimental.pallas import tpu_sc as plsc


def _vq_idx_body(x_ref, w_ref, idx_ref, data_ref, *, imgs_per_step):
    w = w_ref[...]
    w2 = jnp.sum(w * w, axis=1, keepdims=True)
    P = x_ref.shape[2]
    for j in range(imgs_per_step):
        x_t = x_ref[j]
        scores = jax.lax.dot_general(
            w, x_t, (((1,), (0,)), ((), ())),
            preferred_element_type=jnp.float32,
            precision=jax.lax.Precision.DEFAULT,
        )
        d2 = jnp.sum(x_t * x_t, axis=0, keepdims=True)
        dist = d2 - 2.0 * scores + w2
        idx_ref[pl.ds(j * P, P)] = jnp.argmin(dist, axis=0).astype(jnp.int32)
        data_ref[j] = x_t.T


def _make_sc_gather(embed_dim, rows):
    info = plsc.get_sparse_core_info()
    nw = info.num_cores * info.num_subcores
    b_per_w = rows // nw
    mesh = plsc.VectorSubcoreMesh(core_axis_name="c", subcore_axis_name="s")

    @functools.partial(
        pl.kernel, mesh=mesh,
        compiler_params=pltpu.CompilerParams(use_tc_tiling_on_sc=False),
        out_type=jax.ShapeDtypeStruct((rows, embed_dim), jnp.float32),
        scratch_types=[
            pltpu.VMEM((b_per_w,), jnp.int32),
            pltpu.VMEM((b_per_w, embed_dim), jnp.float32),
        ] + [pltpu.SemaphoreType.DMA],
    )
    def gather_k(table_hbm, idx_hbm, out_hbm, idx_v, rows_v, sem):
        wid = lax.axis_index("s") * info.num_cores + lax.axis_index("c")
        base = wid * b_per_w
        pltpu.sync_copy(idx_hbm.at[pl.ds(base, b_per_w)], idx_v)
        pltpu.async_copy(table_hbm.at[idx_v], rows_v, sem).wait()
        pltpu.sync_copy(rows_v, out_hbm.at[pl.ds(base, b_per_w)])

    return gather_k


def kernel(input_data, embed_weights):
    N, C, H, W = input_data.shape
    P = H * W
    rows = N * P
    num_embed = embed_weights.shape[0]
    x = input_data.reshape(N, C, P)
    ips = 4

    idx, data = pl.pallas_call(
        functools.partial(_vq_idx_body, imgs_per_step=ips),
        grid=(N // ips,),
        in_specs=[
            pl.BlockSpec((ips, C, P), lambda i: (i, 0, 0)),
            pl.BlockSpec((num_embed, C), lambda i: (0, 0)),
        ],
        out_specs=[
            pl.BlockSpec((ips * P,), lambda i: (i,)),
            pl.BlockSpec((ips, P, C), lambda i: (i, 0, 0)),
        ],
        out_shape=[
            jax.ShapeDtypeStruct((rows,), jnp.int32),
            jax.ShapeDtypeStruct((N, P, C), jnp.float32),
        ],
    )(x, embed_weights)
    data = data.reshape(rows, C)
    quantize = _make_sc_gather(C, rows)(embed_weights, idx)
    return quantize, quantize, data
